# baseline (speedup 1.0000x reference)
import functools
import math

import jax
import jax.numpy as jnp
from jax import lax
from jax.experimental import pallas as pl
from jax.experimental.pallas import tpu as pltpu
from jax.experimental.pallas import tpu_sc as plsc

N_LIT_C = 20000
N_CLA_C = 10000
E_C = 320000
D_C = 128

_NC = 2
_NS = 16
_KI = 128
_J = 4
_LQ = 40
def _acc_rows(n_dst):
    return ((n_dst + 8 + 127) // 128) * 128

_F32 = jnp.float32


def _nt(x, w):
    return lax.dot_general(x, w, (((1,), (1,)), ((), ())),
                           preferred_element_type=_F32)


def _ln_blk(x, g, b, eps=1e-5):
    m = jnp.mean(x, axis=-1, keepdims=True)
    v = jnp.mean((x - m) * (x - m), axis=-1, keepdims=True)
    return (x - m) * lax.rsqrt(v + eps) * g + b


def _encode_body(inv_sqrt_d, f_ref, w1_ref, b1_ref, w2_ref, b2_ref, o_ref):
    h = jnp.maximum(_nt(f_ref[...], w1_ref[...]) + b1_ref[...], 0.0)
    o_ref[...] = (_nt(h, w2_ref[...]) + b2_ref[...]) * inv_sqrt_d


def _encode(feat, w1, b1, w2, b2, bn):
    n, fdim = feat.shape
    d = w1.shape[0]
    return pl.pallas_call(
        functools.partial(_encode_body, 1.0 / math.sqrt(D_C)),
        grid=(n // bn,),
        in_specs=[
            pl.BlockSpec((bn, fdim), lambda i: (i, 0)),
            pl.BlockSpec((d, fdim), lambda i: (0, 0)),
            pl.BlockSpec((1, d), lambda i: (0, 0)),
            pl.BlockSpec((d, d), lambda i: (0, 0)),
            pl.BlockSpec((1, d), lambda i: (0, 0)),
        ],
        out_specs=pl.BlockSpec((bn, d), lambda i: (i, 0)),
        out_shape=jax.ShapeDtypeStruct((n, d), _F32),
    )(feat, w1, b1, w2, b2)


def _msg_body(h_ref, w_ref, b_ref, o0_ref, o1_ref):
    y = _nt(h_ref[...], w_ref[...]) + b_ref[...]
    o0_ref[...] = y[:, : D_C // 2]
    o1_ref[...] = y[:, D_C // 2:]


def _msg(h, w, b, bn, n_out=None):
    n = h.shape[0]
    n_out = n if n_out is None else n_out
    half = D_C // 2
    return pl.pallas_call(
        _msg_body,
        grid=(n // bn,),
        in_specs=[
            pl.BlockSpec((bn, D_C), lambda i: (i, 0)),
            pl.BlockSpec((D_C, D_C), lambda i: (0, 0)),
            pl.BlockSpec((1, D_C), lambda i: (0, 0)),
        ],
        out_specs=[
            pl.BlockSpec((bn, half), lambda i: (i, 0)),
            pl.BlockSpec((bn, half), lambda i: (i, 0)),
        ],
        out_shape=[
            jax.ShapeDtypeStruct((n_out, half), _F32),
            jax.ShapeDtypeStruct((n_out, half), _F32),
        ],
    )(h, w, b)


def _msg_full_body(h_ref, w_ref, b_ref, o_ref):
    o_ref[...] = _nt(h_ref[...], w_ref[...]) + b_ref[...]


def _msg_full(h, w, b, bn):
    n = h.shape[0]
    return pl.pallas_call(
        _msg_full_body,
        grid=(n // bn,),
        in_specs=[
            pl.BlockSpec((bn, D_C), lambda i: (i, 0)),
            pl.BlockSpec((D_C, D_C), lambda i: (0, 0)),
            pl.BlockSpec((1, D_C), lambda i: (0, 0)),
        ],
        out_specs=pl.BlockSpec((bn, D_C), lambda i: (i, 0)),
        out_shape=jax.ShapeDtypeStruct((n, D_C), _F32),
    )(h, w, b)


def _gates(a, c):
    d = D_C
    i = a[:, :d]
    f = a[:, d:2 * d]
    g = a[:, 2 * d:3 * d]
    o = a[:, 3 * d:]
    cn = jax.nn.sigmoid(f) * c + jax.nn.sigmoid(i) * jnp.tanh(g)
    return cn, jax.nn.sigmoid(o)


def _lstm_cla_body(p0_ref, p1_ref, h_ref, c_ref, wih_ref, bih_ref,
                   whh_ref, bhh_ref, gih_ref, bihn_ref, ghh_ref, bhhn_ref,
                   gc_ref, bcn_ref, hn_ref, cn_ref):
    x = _nt(p0_ref[...] + p1_ref[...], wih_ref[...]) + bih_ref[...]
    a = (_ln_blk(x, gih_ref[...], bihn_ref[...]) +
         _ln_blk(_nt(h_ref[...], whh_ref[...]) + bhh_ref[...], ghh_ref[...], bhhn_ref[...]))
    cn, so = _gates(a, c_ref[...])
    cn_ref[...] = cn
    hn_ref[...] = so * jnp.tanh(_ln_blk(cn, gc_ref[...], bcn_ref[...]))


def _lstm_lit_body(p0_ref, p1_ref, h_ref, c_ref, wa_ref, wb_ref, wf_ref,
                   bih_ref, whh_ref, bhh_ref, gih_ref, bihn_ref, ghh_ref,
                   bhhn_ref, gc_ref, bcn_ref, hn_ref, cn_ref):
    h = h_ref[...]
    bn = h.shape[0]
    up = pltpu.roll(h, bn - 1, 0)
    down = pltpu.roll(h, 1, 0)
    even = (lax.broadcasted_iota(jnp.int32, (bn, 1), 0) % 2) == 0
    flip = jnp.where(even, up, down)
    x = (_nt(p0_ref[...], wa_ref[...]) + _nt(p1_ref[...], wb_ref[...]) +
         _nt(flip, wf_ref[...]) + bih_ref[...])
    a = (_ln_blk(x, gih_ref[...], bihn_ref[...]) +
         _ln_blk(_nt(h, whh_ref[...]) + bhh_ref[...], ghh_ref[...], bhhn_ref[...]))
    cn, so = _gates(a, c_ref[...])
    cn_ref[...] = cn
    hn_ref[...] = so * jnp.tanh(_ln_blk(cn, gc_ref[...], bcn_ref[...]))


def _row_spec(bn, d):
    return pl.BlockSpec((bn, d), lambda i: (i, 0))


def _full_spec(shape):
    return pl.BlockSpec(shape, lambda i: tuple(0 for _ in shape))


def _lstm_call(body, n, bn, args, wspecs, pre_d):
    in_specs = [
        _row_spec(bn, pre_d), _row_spec(bn, pre_d),
        _row_spec(bn, D_C), _row_spec(bn, D_C),
    ] + wspecs
    return pl.pallas_call(
        body,
        grid=(n // bn,),
        in_specs=in_specs,
        out_specs=[_row_spec(bn, D_C), _row_spec(bn, D_C)],
        out_shape=[jax.ShapeDtypeStruct((n, D_C), _F32),
                   jax.ShapeDtypeStruct((n, D_C), _F32)],
    )(*args)


def _seg_tile_loop(msg_ref, acc, src2, dst2, idxs_v, idxd_v, rows_v, sems,
                   sid, rows_per_tile):
    gsems, ssems = sems
    nj = rows_v.shape[0]
    tile0 = sid * rows_per_tile
    outer = rows_per_tile // _LQ
    inner = _LQ // nj

    def outer_body(q, _):
        base = tile0 + q * _LQ

        @pl.when(q > 0)
        def _():
            for j in range(nj):
                pltpu.make_async_copy(rows_v.at[j], acc.at[idxd_v.at[j]],
                                      ssems[j]).wait()

        pltpu.sync_copy(src2.at[pl.ds(base, _LQ)], idxs_v)
        pltpu.sync_copy(dst2.at[pl.ds(base, _LQ)], idxd_v)

        def inner_body(k, _):
            r0 = k * nj

            @pl.when(k > 0)
            def _():
                for j in range(nj):
                    pltpu.make_async_copy(
                        rows_v.at[j], acc.at[idxd_v.at[r0 + j]],
                        ssems[j]).wait()

            for j in range(nj):
                pltpu.async_copy(msg_ref.at[idxs_v.at[r0 + j]],
                                 rows_v.at[j], gsems[j])
            for j in range(nj):
                pltpu.make_async_copy(msg_ref.at[idxs_v.at[r0 + j]],
                                      rows_v.at[j], gsems[j]).wait()
                pltpu.async_copy(rows_v.at[j], acc.at[idxd_v.at[r0 + j]],
                                 ssems[j], add=True)
            return ()

        lax.fori_loop(0, inner, inner_body, (), unroll=False)
        return ()

    lax.fori_loop(0, outer, outer_body, (), unroll=False)
    for j in range(nj):
        pltpu.make_async_copy(rows_v.at[j], acc.at[idxd_v.at[j]],
                              ssems[j]).wait()


def _make_seg(n_src, n_acc, e_pad):
    rpt_acc = n_acc // _NS
    idx_rows = e_pad // _KI
    rows_per_tile = idx_rows // _NS
    half = D_C // 2
    mesh = plsc.VectorSubcoreMesh(core_axis_name="c", subcore_axis_name="s",
                                  num_cores=_NC, num_subcores=_NS)

    @functools.partial(
        pl.kernel,
        out_type=[jax.ShapeDtypeStruct((n_acc, half), _F32),
                  jax.ShapeDtypeStruct((n_acc, half), _F32)],
        mesh=mesh,
        scratch_types=[
            pltpu.VMEM((_LQ, _KI), jnp.int32),
            pltpu.VMEM((_LQ, _KI), jnp.int32),
            pltpu.VMEM((_J, _KI, half), _F32),
            pltpu.VMEM_SHARED((n_acc, half), _F32),
            (tuple(pltpu.SemaphoreType.DMA for _ in range(_J)),
             tuple(pltpu.SemaphoreType.DMA for _ in range(_J))),
        ],
        compiler_params=pltpu.CompilerParams(use_tc_tiling_on_sc=False),
    )
    def seg(msg0, msg1, src2, dst2, zeros_h, out0, out1,
            idxs_v, idxd_v, rows_v, acc, sems):
        cid = lax.axis_index("c")
        sid = lax.axis_index("s")

        pltpu.sync_copy(zeros_h.at[pl.ds(sid * rpt_acc, rpt_acc)],
                        acc.at[pl.ds(sid * rpt_acc, rpt_acc)])
        plsc.subcore_barrier()

        @pl.when(cid == 0)
        def _():
            _seg_tile_loop(msg0, acc, src2, dst2, idxs_v, idxd_v, rows_v,
                           sems, sid, rows_per_tile)

        @pl.when(cid == 1)
        def _():
            _seg_tile_loop(msg1, acc, src2, dst2, idxs_v, idxd_v, rows_v,
                           sems, sid, rows_per_tile)

        plsc.subcore_barrier()

        @pl.when(cid == 0)
        def _():
            pltpu.sync_copy(acc.at[pl.ds(sid * rpt_acc, rpt_acc)],
                            out0.at[pl.ds(sid * rpt_acc, rpt_acc)])

        @pl.when(cid == 1)
        def _():
            pltpu.sync_copy(acc.at[pl.ds(sid * rpt_acc, rpt_acc)],
                            out1.at[pl.ds(sid * rpt_acc, rpt_acc)])

    return seg


def _make_seg_full(n_acc, e_pad):
    rpt_acc = n_acc // _NS
    idx_rows = e_pad // _KI
    rows_per_worker = idx_rows // (_NC * _NS)
    njf = 2
    mesh = plsc.VectorSubcoreMesh(core_axis_name="c", subcore_axis_name="s",
                                  num_cores=_NC, num_subcores=_NS)

    @functools.partial(
        pl.kernel,
        out_type=[jax.ShapeDtypeStruct((n_acc, D_C), _F32),
                  jax.ShapeDtypeStruct((n_acc, D_C), _F32)],
        mesh=mesh,
        scratch_types=[
            pltpu.VMEM((_LQ, _KI), jnp.int32),
            pltpu.VMEM((_LQ, _KI), jnp.int32),
            pltpu.VMEM((njf, _KI, D_C), _F32),
            pltpu.VMEM_SHARED((n_acc, D_C), _F32),
            (tuple(pltpu.SemaphoreType.DMA for _ in range(njf)),
             tuple(pltpu.SemaphoreType.DMA for _ in range(njf))),
        ],
        compiler_params=pltpu.CompilerParams(use_tc_tiling_on_sc=False),
    )
    def seg(msg, src2, dst2, zeros_h, out0, out1,
            idxs_v, idxd_v, rows_v, acc, sems):
        cid = lax.axis_index("c")
        sid = lax.axis_index("s")
        wid = cid * _NS + sid

        pltpu.sync_copy(zeros_h.at[pl.ds(sid * rpt_acc, rpt_acc)],
                        acc.at[pl.ds(sid * rpt_acc, rpt_acc)])
        plsc.subcore_barrier()

        _seg_tile_loop(msg, acc, src2, dst2, idxs_v, idxd_v, rows_v,
                       sems, wid, rows_per_worker)

        plsc.subcore_barrier()

        @pl.when(cid == 0)
        def _():
            pltpu.sync_copy(acc.at[pl.ds(sid * rpt_acc, rpt_acc)],
                            out0.at[pl.ds(sid * rpt_acc, rpt_acc)])

        @pl.when(cid == 1)
        def _():
            pltpu.sync_copy(acc.at[pl.ds(sid * rpt_acc, rpt_acc)],
                            out1.at[pl.ds(sid * rpt_acc, rpt_acc)])

    return seg


def _pad_edges(src, dst, n_dst, e_pad):
    npad = e_pad - src.shape[0]
    src_p = jnp.concatenate([src.astype(jnp.int32),
                             jnp.zeros((npad,), jnp.int32)])
    dst_p = jnp.concatenate([dst.astype(jnp.int32),
                             n_dst + (jnp.arange(npad, dtype=jnp.int32) % 8)])
    return src_p.reshape(-1, _KI), dst_p.reshape(-1, _KI)


def kernel(literal_feat, clause_feat, edge_lit, edge_clause, Wl1, bl1, Wl2,
           bl2, Wc1, bc1, Wc2, bc2, Wlc, blc, Wcl, bcl, L_Wih, L_bih, L_Whh,
           L_bhh, L_g_ih, L_b_ih, L_g_hh, L_b_hh, L_g_c, L_b_c, C_Wih, C_bih,
           C_Whh, C_bhh, C_g_ih, C_b_ih, C_g_hh, C_b_hh, C_g_c, C_b_c):
    d = D_C
    half = d // 2
    row = lambda v: v.reshape(1, -1)

    e_group = _NS * _J * _KI
    e_pad = ((E_C + e_group - 1) // e_group) * e_group
    sl2c, dl2c = _pad_edges(edge_lit, edge_clause, N_CLA_C, e_pad)
    sc2l, dc2l = _pad_edges(edge_clause, edge_lit, N_LIT_C, e_pad)
    n_acc_l = _acc_rows(N_LIT_C)
    n_acc_c = _acc_rows(N_CLA_C)
    z = jnp.zeros((n_acc_l, half), _F32)
    zc = jnp.zeros((n_acc_c, d), _F32)

    seg = _make_seg(N_LIT_C, n_acc_l, e_pad)
    seg_f = _make_seg_full(n_acc_c, e_pad)

    lit = _encode(literal_feat, Wl1, row(bl1), Wl2, row(bl2), 1000)
    cla = _encode(clause_feat, Wc1, row(bc1), Wc2, row(bc2), 1000)

    Lh, Lc = lit, jnp.zeros_like(lit)
    Ch, Cc = cla, jnp.zeros_like(cla)

    L_WihA = L_Wih[:, :half]
    L_WihB = L_Wih[:, half:d]
    L_WihF = L_Wih[:, d:]

    tail = [
        _full_spec((1, 4 * d)), _full_spec((4 * d, d)), _full_spec((1, 4 * d)),
        _full_spec((1, 4 * d)), _full_spec((1, 4 * d)),
        _full_spec((1, 4 * d)), _full_spec((1, 4 * d)),
        _full_spec((1, d)), _full_spec((1, d)),
    ]
    wspec_cla = [_full_spec((4 * d, d))] + tail
    wspec_lit = ([_full_spec((4 * d, half)), _full_spec((4 * d, half)),
                  _full_spec((4 * d, d))] + tail)

    for _ in range(2):
        mf = _msg_full(Lh, Wlc, row(blc), 1000)
        pa, pb = seg_f(mf, sl2c, dl2c, zc)
        pa, pb = pa[:N_CLA_C], pb[:N_CLA_C]
        Ch, Cc = _lstm_call(
            _lstm_cla_body, N_CLA_C, 1000,
            (pa, pb, Ch, Cc, C_Wih, row(C_bih), C_Whh, row(C_bhh),
             row(C_g_ih), row(C_b_ih), row(C_g_hh), row(C_b_hh), row(C_g_c),
             row(C_b_c)),
            wspec_cla, d)
        m0, m1 = _msg(Ch, Wcl, row(bcl), 1000, n_out=N_LIT_C)
        q0, q1 = seg(m0, m1, sc2l, dc2l, z)
        q0, q1 = q0[:N_LIT_C], q1[:N_LIT_C]
        Lh, Lc = _lstm_call(
            _lstm_lit_body, N_LIT_C, 1000,
            (q0, q1, Lh, Lc, L_WihA, L_WihB, L_WihF, row(L_bih), L_Whh,
             row(L_bhh), row(L_g_ih), row(L_b_ih), row(L_g_hh), row(L_b_hh),
             row(L_g_c), row(L_b_c)),
            wspec_lit, half)

    return (Lh, Ch)


# speedup vs baseline: 1.2367x; 1.2367x over previous
import functools
import math

import jax
import jax.numpy as jnp
from jax import lax
from jax.experimental import pallas as pl
from jax.experimental.pallas import tpu as pltpu
from jax.experimental.pallas import tpu_sc as plsc

N_LIT_C = 20000
N_CLA_C = 10000
E_C = 320000
D_C = 128

_NC = 2
_NS = 16
_KI = 128
_J = 4
_LQ = 40
def _acc_rows(n_dst):
    return ((n_dst + 8 + 127) // 128) * 128

_F32 = jnp.float32


def _nt(x, w):
    return lax.dot_general(x, w, (((1,), (1,)), ((), ())),
                           preferred_element_type=_F32)


def _ln_blk(x, g, b, eps=1e-5):
    m = jnp.mean(x, axis=-1, keepdims=True)
    v = jnp.mean((x - m) * (x - m), axis=-1, keepdims=True)
    return (x - m) * lax.rsqrt(v + eps) * g + b


def _encode_body(inv_sqrt_d, f_ref, w1_ref, b1_ref, w2_ref, b2_ref, o_ref):
    h = jnp.maximum(_nt(f_ref[...], w1_ref[...]) + b1_ref[...], 0.0)
    o_ref[...] = (_nt(h, w2_ref[...]) + b2_ref[...]) * inv_sqrt_d


def _encode_msg_body(inv_sqrt_d, f_ref, w1_ref, b1_ref, w2_ref, b2_ref,
                     wm_ref, bm_ref, o_ref, m0_ref, m1_ref):
    h = jnp.maximum(_nt(f_ref[...], w1_ref[...]) + b1_ref[...], 0.0)
    o = (_nt(h, w2_ref[...]) + b2_ref[...]) * inv_sqrt_d
    o_ref[...] = o
    y = _nt(o, wm_ref[...]) + bm_ref[...]
    m0_ref[...] = y[:, : D_C // 2]
    m1_ref[...] = y[:, D_C // 2:]


def _encode_msg(feat, w1, b1, w2, b2, wm, bm, bn):
    n, fdim = feat.shape
    d = w1.shape[0]
    half = d // 2
    return pl.pallas_call(
        functools.partial(_encode_msg_body, 1.0 / math.sqrt(D_C)),
        grid=(n // bn,),
        in_specs=[
            pl.BlockSpec((bn, fdim), lambda i: (i, 0)),
            pl.BlockSpec((d, fdim), lambda i: (0, 0)),
            pl.BlockSpec((1, d), lambda i: (0, 0)),
            pl.BlockSpec((d, d), lambda i: (0, 0)),
            pl.BlockSpec((1, d), lambda i: (0, 0)),
            pl.BlockSpec((d, d), lambda i: (0, 0)),
            pl.BlockSpec((1, d), lambda i: (0, 0)),
        ],
        out_specs=[pl.BlockSpec((bn, d), lambda i: (i, 0)),
                   pl.BlockSpec((bn, half), lambda i: (i, 0)),
                   pl.BlockSpec((bn, half), lambda i: (i, 0))],
        out_shape=[jax.ShapeDtypeStruct((n, d), _F32),
                   jax.ShapeDtypeStruct((n, half), _F32),
                   jax.ShapeDtypeStruct((n, half), _F32)],
    )(feat, w1, b1, w2, b2, wm, bm)


def _encode(feat, w1, b1, w2, b2, bn):
    n, fdim = feat.shape
    d = w1.shape[0]
    return pl.pallas_call(
        functools.partial(_encode_body, 1.0 / math.sqrt(D_C)),
        grid=(n // bn,),
        in_specs=[
            pl.BlockSpec((bn, fdim), lambda i: (i, 0)),
            pl.BlockSpec((d, fdim), lambda i: (0, 0)),
            pl.BlockSpec((1, d), lambda i: (0, 0)),
            pl.BlockSpec((d, d), lambda i: (0, 0)),
            pl.BlockSpec((1, d), lambda i: (0, 0)),
        ],
        out_specs=pl.BlockSpec((bn, d), lambda i: (i, 0)),
        out_shape=jax.ShapeDtypeStruct((n, d), _F32),
    )(feat, w1, b1, w2, b2)


def _msg_body(h_ref, w_ref, b_ref, o0_ref, o1_ref):
    y = _nt(h_ref[...], w_ref[...]) + b_ref[...]
    o0_ref[...] = y[:, : D_C // 2]
    o1_ref[...] = y[:, D_C // 2:]


def _msg(h, w, b, bn, n_out=None):
    n = h.shape[0]
    n_out = n if n_out is None else n_out
    half = D_C // 2
    return pl.pallas_call(
        _msg_body,
        grid=(n // bn,),
        in_specs=[
            pl.BlockSpec((bn, D_C), lambda i: (i, 0)),
            pl.BlockSpec((D_C, D_C), lambda i: (0, 0)),
            pl.BlockSpec((1, D_C), lambda i: (0, 0)),
        ],
        out_specs=[
            pl.BlockSpec((bn, half), lambda i: (i, 0)),
            pl.BlockSpec((bn, half), lambda i: (i, 0)),
        ],
        out_shape=[
            jax.ShapeDtypeStruct((n_out, half), _F32),
            jax.ShapeDtypeStruct((n_out, half), _F32),
        ],
    )(h, w, b)


def _gates(a, c):
    d = D_C
    i = a[:, :d]
    f = a[:, d:2 * d]
    g = a[:, 2 * d:3 * d]
    o = a[:, 3 * d:]
    cn = jax.nn.sigmoid(f) * c + jax.nn.sigmoid(i) * jnp.tanh(g)
    return cn, jax.nn.sigmoid(o)


def _lstm_cla_body(p0_ref, p1_ref, h_ref, c_ref, wa_ref, wb_ref, bih_ref,
                   whh_ref, bhh_ref, gih_ref, bihn_ref, ghh_ref, bhhn_ref,
                   gc_ref, bcn_ref, wm_ref, bm_ref, hn_ref, cn_ref,
                   m0_ref, m1_ref):
    x = _nt(p0_ref[...], wa_ref[...]) + _nt(p1_ref[...], wb_ref[...]) + bih_ref[...]
    a = (_ln_blk(x, gih_ref[...], bihn_ref[...]) +
         _ln_blk(_nt(h_ref[...], whh_ref[...]) + bhh_ref[...], ghh_ref[...], bhhn_ref[...]))
    cn, so = _gates(a, c_ref[...])
    hn = so * jnp.tanh(_ln_blk(cn, gc_ref[...], bcn_ref[...]))
    cn_ref[...] = cn
    hn_ref[...] = hn
    y = _nt(hn, wm_ref[...]) + bm_ref[...]
    m0_ref[...] = y[:, : D_C // 2]
    m1_ref[...] = y[:, D_C // 2:]


def _lstm_lit_body(p0_ref, p1_ref, h_ref, c_ref, wa_ref, wb_ref, wf_ref,
                   bih_ref, whh_ref, bhh_ref, gih_ref, bihn_ref, ghh_ref,
                   bhhn_ref, gc_ref, bcn_ref, wm_ref, bm_ref, hn_ref, cn_ref,
                   m0_ref, m1_ref):
    h = h_ref[...]
    bn = h.shape[0]
    up = pltpu.roll(h, bn - 1, 0)
    down = pltpu.roll(h, 1, 0)
    even = (lax.broadcasted_iota(jnp.int32, (bn, 1), 0) % 2) == 0
    flip = jnp.where(even, up, down)
    x = (_nt(p0_ref[...], wa_ref[...]) + _nt(p1_ref[...], wb_ref[...]) +
         _nt(flip, wf_ref[...]) + bih_ref[...])
    a = (_ln_blk(x, gih_ref[...], bihn_ref[...]) +
         _ln_blk(_nt(h, whh_ref[...]) + bhh_ref[...], ghh_ref[...], bhhn_ref[...]))
    cn, so = _gates(a, c_ref[...])
    hn = so * jnp.tanh(_ln_blk(cn, gc_ref[...], bcn_ref[...]))
    cn_ref[...] = cn
    hn_ref[...] = hn
    y = _nt(hn, wm_ref[...]) + bm_ref[...]
    m0_ref[...] = y[:, : D_C // 2]
    m1_ref[...] = y[:, D_C // 2:]


def _row_spec(bn, d):
    return pl.BlockSpec((bn, d), lambda i: (i, 0))


def _full_spec(shape):
    return pl.BlockSpec(shape, lambda i: tuple(0 for _ in shape))


def _lstm_call(body, n, bn, args, wspecs, n_msg):
    half = D_C // 2
    in_specs = [
        _row_spec(bn, half), _row_spec(bn, half),
        _row_spec(bn, D_C), _row_spec(bn, D_C),
    ] + wspecs + [_full_spec((D_C, D_C)), _full_spec((1, D_C))]
    return pl.pallas_call(
        body,
        grid=(n // bn,),
        in_specs=in_specs,
        out_specs=[_row_spec(bn, D_C), _row_spec(bn, D_C),
                   _row_spec(bn, half), _row_spec(bn, half)],
        out_shape=[jax.ShapeDtypeStruct((n, D_C), _F32),
                   jax.ShapeDtypeStruct((n, D_C), _F32),
                   jax.ShapeDtypeStruct((n_msg, half), _F32),
                   jax.ShapeDtypeStruct((n_msg, half), _F32)],
    )(*args)


def _seg_tile_loop(msg_ref, acc, src2, dst2, idxs_v, idxd_v, rows_v, sems,
                   sid, rows_per_tile):
    gsems, ssems = sems
    tile0 = sid * rows_per_tile
    outer = rows_per_tile // _LQ
    inner = _LQ // _J

    def outer_body(q, _):
        base = tile0 + q * _LQ

        @pl.when(q > 0)
        def _():
            for j in range(_J):
                pltpu.make_async_copy(rows_v.at[j], acc.at[idxd_v.at[j]],
                                      ssems[j]).wait()

        pltpu.sync_copy(src2.at[pl.ds(base, _LQ)], idxs_v)
        pltpu.sync_copy(dst2.at[pl.ds(base, _LQ)], idxd_v)

        def inner_body(k, _):
            r0 = k * _J

            @pl.when(k > 0)
            def _():
                for j in range(_J):
                    pltpu.make_async_copy(
                        rows_v.at[j], acc.at[idxd_v.at[r0 + j]],
                        ssems[j]).wait()

            for j in range(_J):
                pltpu.async_copy(msg_ref.at[idxs_v.at[r0 + j]],
                                 rows_v.at[j], gsems[j])
            for j in range(_J):
                pltpu.make_async_copy(msg_ref.at[idxs_v.at[r0 + j]],
                                      rows_v.at[j], gsems[j]).wait()
                pltpu.async_copy(rows_v.at[j], acc.at[idxd_v.at[r0 + j]],
                                 ssems[j], add=True)
            return ()

        lax.fori_loop(0, inner, inner_body, (), unroll=False)
        return ()

    lax.fori_loop(0, outer, outer_body, (), unroll=False)
    for j in range(_J):
        pltpu.make_async_copy(rows_v.at[j], acc.at[idxd_v.at[j]],
                              ssems[j]).wait()


def _make_seg(n_src, n_acc, e_pad):
    rpt_acc = n_acc // _NS
    idx_rows = e_pad // _KI
    rows_per_tile = idx_rows // _NS
    half = D_C // 2
    mesh = plsc.VectorSubcoreMesh(core_axis_name="c", subcore_axis_name="s",
                                  num_cores=_NC, num_subcores=_NS)

    @functools.partial(
        pl.kernel,
        out_type=[jax.ShapeDtypeStruct((n_acc, half), _F32),
                  jax.ShapeDtypeStruct((n_acc, half), _F32)],
        mesh=mesh,
        scratch_types=[
            pltpu.VMEM((_LQ, _KI), jnp.int32),
            pltpu.VMEM((_LQ, _KI), jnp.int32),
            pltpu.VMEM((_J, _KI, half), _F32),
            pltpu.VMEM_SHARED((n_acc, half), _F32),
            (tuple(pltpu.SemaphoreType.DMA for _ in range(_J)),
             tuple(pltpu.SemaphoreType.DMA for _ in range(_J))),
        ],
        compiler_params=pltpu.CompilerParams(use_tc_tiling_on_sc=False),
    )
    def seg(msg0, msg1, src2, dst2, zeros_h, out0, out1,
            idxs_v, idxd_v, rows_v, acc, sems):
        cid = lax.axis_index("c")
        sid = lax.axis_index("s")

        pltpu.sync_copy(zeros_h.at[pl.ds(sid * rpt_acc, rpt_acc)],
                        acc.at[pl.ds(sid * rpt_acc, rpt_acc)])
        plsc.subcore_barrier()

        @pl.when(cid == 0)
        def _():
            _seg_tile_loop(msg0, acc, src2, dst2, idxs_v, idxd_v, rows_v,
                           sems, sid, rows_per_tile)

        @pl.when(cid == 1)
        def _():
            _seg_tile_loop(msg1, acc, src2, dst2, idxs_v, idxd_v, rows_v,
                           sems, sid, rows_per_tile)

        plsc.subcore_barrier()

        @pl.when(cid == 0)
        def _():
            pltpu.sync_copy(acc.at[pl.ds(sid * rpt_acc, rpt_acc)],
                            out0.at[pl.ds(sid * rpt_acc, rpt_acc)])

        @pl.when(cid == 1)
        def _():
            pltpu.sync_copy(acc.at[pl.ds(sid * rpt_acc, rpt_acc)],
                            out1.at[pl.ds(sid * rpt_acc, rpt_acc)])

    return seg


def _pad_edges(src, dst, n_dst, e_pad):
    npad = e_pad - src.shape[0]
    src_p = jnp.concatenate([src.astype(jnp.int32),
                             jnp.zeros((npad,), jnp.int32)])
    dst_p = jnp.concatenate([dst.astype(jnp.int32),
                             n_dst + (jnp.arange(npad, dtype=jnp.int32) % 8)])
    return src_p.reshape(-1, _KI), dst_p.reshape(-1, _KI)


def kernel(literal_feat, clause_feat, edge_lit, edge_clause, Wl1, bl1, Wl2,
           bl2, Wc1, bc1, Wc2, bc2, Wlc, blc, Wcl, bcl, L_Wih, L_bih, L_Whh,
           L_bhh, L_g_ih, L_b_ih, L_g_hh, L_b_hh, L_g_c, L_b_c, C_Wih, C_bih,
           C_Whh, C_bhh, C_g_ih, C_b_ih, C_g_hh, C_b_hh, C_g_c, C_b_c):
    d = D_C
    half = d // 2
    row = lambda v: v.reshape(1, -1)

    e_group = _NS * _J * _KI
    e_pad = ((E_C + e_group - 1) // e_group) * e_group
    sl2c, dl2c = _pad_edges(edge_lit, edge_clause, N_CLA_C, e_pad)
    sc2l, dc2l = _pad_edges(edge_clause, edge_lit, N_LIT_C, e_pad)
    n_acc = _acc_rows(N_LIT_C)
    z = jnp.zeros((n_acc, half), _F32)

    seg = _make_seg(N_LIT_C, n_acc, e_pad)

    lit, lm0, lm1 = _encode_msg(literal_feat, Wl1, row(bl1), Wl2, row(bl2),
                                Wlc, row(blc), 1000)
    cla = _encode(clause_feat, Wc1, row(bc1), Wc2, row(bc2), 1000)

    Lh, Lc = lit, jnp.zeros_like(lit)
    Ch, Cc = cla, jnp.zeros_like(cla)

    C_WihA = C_Wih[:, :half]
    C_WihB = C_Wih[:, half:]
    L_WihA = L_Wih[:, :half]
    L_WihB = L_Wih[:, half:d]
    L_WihF = L_Wih[:, d:]

    wspec_cla = [
        _full_spec((4 * d, half)), _full_spec((4 * d, half)),
        _full_spec((1, 4 * d)), _full_spec((4 * d, d)), _full_spec((1, 4 * d)),
        _full_spec((1, 4 * d)), _full_spec((1, 4 * d)),
        _full_spec((1, 4 * d)), _full_spec((1, 4 * d)),
        _full_spec((1, d)), _full_spec((1, d)),
    ]
    wspec_lit = ([_full_spec((4 * d, half)), _full_spec((4 * d, half)),
                  _full_spec((4 * d, d))] + wspec_cla[2:])

    for _ in range(2):
        p0, p1 = seg(lm0, lm1, sl2c, dl2c, z)
        Ch, Cc, cm0, cm1 = _lstm_call(
            _lstm_cla_body, N_CLA_C, 1000,
            (p0, p1, Ch, Cc, C_WihA, C_WihB, row(C_bih), C_Whh, row(C_bhh),
             row(C_g_ih), row(C_b_ih), row(C_g_hh), row(C_b_hh), row(C_g_c),
             row(C_b_c), Wcl, row(bcl)),
            wspec_cla, N_LIT_C)
        q0, q1 = seg(cm0, cm1, sc2l, dc2l, z)
        Lh, Lc, lm0, lm1 = _lstm_call(
            _lstm_lit_body, N_LIT_C, 1000,
            (q0, q1, Lh, Lc, L_WihA, L_WihB, L_WihF, row(L_bih), L_Whh,
             row(L_bhh), row(L_g_ih), row(L_b_ih), row(L_g_hh), row(L_b_hh),
             row(L_g_c), row(L_b_c), Wlc, row(blc)),
            wspec_lit, N_LIT_C)

    return (Lh, Ch)
